# baseline (speedup 1.0000x reference)
import functools

import jax
import jax.numpy as jnp
from jax import lax
from jax.experimental import pallas as pl
from jax.experimental.pallas import tpu as pltpu
from jax.experimental.pallas import tpu_sc as plsc

VOCAB = 1000000
LATENT = 32
BATCH = 4096
SEQ = 200

NC = 2
NS = 16
NW = NC * NS

BPW = BATCH // NW
G = LATENT // 8
NBUF = 2
NBLK = SEQ // NBUF


def _make_gather():
  mesh = plsc.VectorSubcoreMesh(core_axis_name="c", subcore_axis_name="s")

  @functools.partial(
      pl.kernel,
      out_type=jax.ShapeDtypeStruct((SEQ, G, NW, 8, BPW), jnp.float32),
      mesh=mesh,
      compiler_params=pltpu.CompilerParams(use_tc_tiling_on_sc=False, needs_layout_passes=False),
      scratch_types=[
          pltpu.VMEM((SEQ, BPW), jnp.int32),
          [pltpu.VMEM((BPW, LATENT), jnp.float32) for _ in range(NBUF)],
          [pltpu.VMEM((G, 8, BPW), jnp.float32) for _ in range(NBUF)],
          [pltpu.SemaphoreType.DMA for _ in range(NBUF)],
          [pltpu.SemaphoreType.DMA for _ in range(NBUF)],
      ],
  )
  def gather_kernel(table_hbm, idst_hbm, out_hbm, idx_v, gbufs, tbufs,
                    gsems, ssems):
    wid = lax.axis_index("s") * NC + lax.axis_index("c")
    pltpu.sync_copy(idst_hbm.at[:, pl.ds(wid * BPW, BPW)], idx_v)

    lane = lax.iota(jnp.int32, 16)

    def fire(s, b):
      pltpu.async_copy(table_hbm.at[idx_v.at[s]], gbufs[b], gsems[b])

    def drain_gather(b):
      pltpu.make_async_copy(
          table_hbm.at[pl.ds(0, BPW)], gbufs[b], gsems[b]
      ).wait()

    def transpose(b):
      gb, tb = gbufs[b], tbufs[b]
      for g in range(G):
        for r in range(8):
          col_idx = jnp.full((16,), 8 * g + r, jnp.int32)
          for k in range(BPW // 16):
            vals = plsc.load_gather(gb, [lane + 16 * k, col_idx])
            tb[g, r, pl.ds(16 * k, 16)] = vals

    def fire_store(s, b):
      pltpu.async_copy(tbufs[b], out_hbm.at[s, :, wid], ssems[b])

    def drain_store(b):
      pltpu.make_async_copy(
          out_hbm.at[0, :, wid], tbufs[b], ssems[b]
      ).wait()

    for b in range(NBUF):
      fire(b, b)

    for b in range(NBUF):
      drain_gather(b)
      transpose(b)
      fire_store(b, b)
      fire(b + NBUF, b)

    def block(blk, _):
      for b in range(NBUF):
        s = blk * NBUF + b
        drain_gather(b)
        drain_store(b)
        transpose(b)
        fire_store(s, b)
        fire(s + NBUF, b)
      return _

    lax.fori_loop(1, NBLK - 1, block, None)

    for b in range(NBUF):
      s = (NBLK - 1) * NBUF + b
      drain_gather(b)
      drain_store(b)
      transpose(b)
      fire_store(s, b)

    for b in range(NBUF):
      drain_store(b)

  return gather_kernel


_gather = _make_gather()


@jax.jit
def kernel(input_ids, table):
  ids_t = input_ids.T.astype(jnp.int32)
  out5 = _gather(table, ids_t)
  out = out5.transpose(2, 4, 0, 1, 3).reshape(BATCH, SEQ, LATENT)
  return out


# speedup vs baseline: 1.1940x; 1.1940x over previous
import functools

import jax
import jax.numpy as jnp
from jax import lax
from jax.experimental import pallas as pl
from jax.experimental.pallas import tpu as pltpu
from jax.experimental.pallas import tpu_sc as plsc

VOCAB = 1000000
LATENT = 32
BATCH = 4096
SEQ = 200

NC = 2
NS = 16
NW = NC * NS

BPW = BATCH // NW
G = LATENT // 8
NBUF = 2
NBLK = SEQ // NBUF


def _make_gather():
  mesh = plsc.VectorSubcoreMesh(core_axis_name="c", subcore_axis_name="s")

  @functools.partial(
      pl.kernel,
      out_type=jax.ShapeDtypeStruct((SEQ, G, NW, 8, BPW), jnp.float32),
      mesh=mesh,
      compiler_params=pltpu.CompilerParams(
          use_tc_tiling_on_sc=False,
          needs_layout_passes=False,
          disable_bounds_checks=True,
      ),
      scratch_types=[
          pltpu.VMEM((SEQ, BPW), jnp.int32),
          [pltpu.VMEM((BPW, LATENT), jnp.float32) for _ in range(NBUF)],
          [pltpu.VMEM((G, 8, BPW), jnp.float32) for _ in range(NBUF)],
          [pltpu.SemaphoreType.DMA for _ in range(NBUF)],
          [pltpu.SemaphoreType.DMA for _ in range(NBUF)],
      ],
  )
  def gather_kernel(table_hbm, idst_hbm, out_hbm, idx_v, gbufs, tbufs,
                    gsems, ssems):
    wid = lax.axis_index("s") * NC + lax.axis_index("c")
    pltpu.sync_copy(idst_hbm.at[:, pl.ds(wid * BPW, BPW)], idx_v)

    lane = lax.iota(jnp.int32, 16)

    def fire(s, b):
      pltpu.async_copy(table_hbm.at[idx_v.at[s]], gbufs[b], gsems[b])

    def drain_gather(b):
      pltpu.make_async_copy(
          table_hbm.at[pl.ds(0, BPW)], gbufs[b], gsems[b]
      ).wait()

    def transpose(b):
      gb, tb = gbufs[b], tbufs[b]
      for g in range(G):
        for r in range(8):
          col_idx = jnp.full((16,), 8 * g + r, jnp.int32)
          vals = [
              plsc.load_gather(gb, [lane + 16 * k, col_idx])
              for k in range(BPW // 16)
          ]
          for k in range(BPW // 16):
            tb[g, r, pl.ds(16 * k, 16)] = vals[k]

    def fire_store(s, b):
      pltpu.async_copy(tbufs[b], out_hbm.at[s, :, wid], ssems[b])

    def drain_store(b):
      pltpu.make_async_copy(
          out_hbm.at[0, :, wid], tbufs[b], ssems[b]
      ).wait()

    for b in range(NBUF):
      fire(b, b)

    for b in range(NBUF):
      drain_gather(b)
      transpose(b)
      fire_store(b, b)
      fire(b + NBUF, b)

    def block(blk, _):
      for b in range(NBUF):
        s = blk * NBUF + b
        drain_gather(b)
        drain_store(b)
        transpose(b)
        fire_store(s, b)
        fire(s + NBUF, b)
      return _

    lax.fori_loop(1, NBLK - 1, block, None)

    for b in range(NBUF):
      s = (NBLK - 1) * NBUF + b
      drain_gather(b)
      drain_store(b)
      transpose(b)
      fire_store(s, b)

    for b in range(NBUF):
      drain_store(b)

  return gather_kernel


_gather = _make_gather()


@jax.jit
def kernel(input_ids, table):
  ids_t = input_ids.T.astype(jnp.int32)
  out5 = _gather(table, ids_t)
  out = out5.transpose(2, 4, 0, 1, 3).reshape(BATCH, SEQ, LATENT)
  return out


# speedup vs baseline: 1.5431x; 1.2924x over previous
import functools

import jax
import jax.numpy as jnp
from jax import lax
from jax.experimental import pallas as pl
from jax.experimental.pallas import tpu as pltpu
from jax.experimental.pallas import tpu_sc as plsc

VOCAB = 1000000
LATENT = 32
BATCH = 4096
SEQ = 200

NC = 2
NS = 16
NW = NC * NS

BPW = BATCH // NW
G = LATENT // 8
NBUF = 2
NBLK = SEQ // NBUF


def _make_gather():
  mesh = plsc.VectorSubcoreMesh(core_axis_name="c", subcore_axis_name="s")

  @functools.partial(
      pl.kernel,
      out_type=jax.ShapeDtypeStruct((SEQ, G, NW, 8 * BPW), jnp.float32),
      mesh=mesh,
      compiler_params=pltpu.CompilerParams(
          use_tc_tiling_on_sc=False,
          needs_layout_passes=False,
          disable_bounds_checks=True,
      ),
      scratch_types=[
          pltpu.VMEM((SEQ, BPW), jnp.int32),
          [pltpu.VMEM((BPW, LATENT), jnp.float32) for _ in range(NBUF)],
          [pltpu.VMEM((G * 8 * BPW,), jnp.float32) for _ in range(NBUF)],
          [pltpu.SemaphoreType.DMA for _ in range(NBUF)],
          [pltpu.SemaphoreType.DMA for _ in range(NBUF)],
      ],
  )
  def gather_kernel(table_hbm, idst_hbm, out_hbm, idx_v, gbufs, tbufs,
                    gsems, ssems):
    wid = lax.axis_index("s") * NC + lax.axis_index("c")
    pltpu.sync_copy(idst_hbm.at[:, pl.ds(wid * BPW, BPW)], idx_v)

    lane = lax.iota(jnp.int32, 16)

    def fire(s, b):
      pltpu.async_copy(table_hbm.at[idx_v.at[s]], gbufs[b], gsems[b])

    def drain_gather(b):
      pltpu.make_async_copy(
          table_hbm.at[pl.ds(0, BPW)], gbufs[b], gsems[b]
      ).wait()

    def transpose(b):
      gb, tb = gbufs[b], tbufs[b]
      for d in range(LATENT):
        q_vec = lax.rem(lane + d, jnp.int32(LATENT))
        flat_base = q_vec * BPW + lane
        vals = [
            plsc.load_gather(gb, [lane + 16 * k, q_vec])
            for k in range(BPW // 16)
        ]
        for k in range(BPW // 16):
          plsc.store_scatter(tb, [flat_base + 16 * k], vals[k])

    def fire_store(s, b):
      for g in range(G):
        pltpu.async_copy(tbufs[b].at[pl.ds(8 * BPW * g, 8 * BPW)],
                         out_hbm.at[s, g, wid], ssems[b])

    def drain_store(b):
      for g in range(G):
        pltpu.make_async_copy(
            out_hbm.at[0, g, wid],
            tbufs[b].at[pl.ds(8 * BPW * g, 8 * BPW)], ssems[b]
        ).wait()

    for b in range(NBUF):
      fire(b, b)

    for b in range(NBUF):
      drain_gather(b)
      transpose(b)
      fire_store(b, b)
      fire(b + NBUF, b)

    def block(blk, _):
      for b in range(NBUF):
        s = blk * NBUF + b
        drain_gather(b)
        drain_store(b)
        transpose(b)
        fire_store(s, b)
        fire(s + NBUF, b)
      return _

    lax.fori_loop(1, NBLK - 1, block, None)

    for b in range(NBUF):
      s = (NBLK - 1) * NBUF + b
      drain_gather(b)
      drain_store(b)
      transpose(b)
      fire_store(s, b)

    for b in range(NBUF):
      drain_store(b)

  return gather_kernel


_gather = _make_gather()


@jax.jit
def kernel(input_ids, table):
  ids_t = input_ids.T.astype(jnp.int32)
  out5 = _gather(table, ids_t).reshape(SEQ, G, NW, 8, BPW)
  out = out5.transpose(2, 4, 0, 1, 3).reshape(BATCH, SEQ, LATENT)
  return out


# speedup vs baseline: 1.8272x; 1.1841x over previous
import functools

import jax
import jax.numpy as jnp
from jax import lax
from jax.experimental import pallas as pl
from jax.experimental.pallas import tpu as pltpu
from jax.experimental.pallas import tpu_sc as plsc

VOCAB = 1000000
LATENT = 32
BATCH = 4096
SEQ = 200

NC = 2
NS = 16
NW = NC * NS

BPW = BATCH // NW
G = LATENT // 8
NBUF = 2
NBLK = SEQ // NBUF


def _make_gather():
  mesh = plsc.VectorSubcoreMesh(core_axis_name="c", subcore_axis_name="s")

  @functools.partial(
      pl.kernel,
      out_type=jax.ShapeDtypeStruct((SEQ, G, NW, 8 * BPW), jnp.float32),
      mesh=mesh,
      compiler_params=pltpu.CompilerParams(
          use_tc_tiling_on_sc=False,
          needs_layout_passes=False,
          disable_bounds_checks=True,
      ),
      scratch_types=[
          pltpu.VMEM((SEQ, BPW), jnp.int32),
          [pltpu.VMEM((BPW, LATENT), jnp.float32) for _ in range(NBUF)],
          [pltpu.VMEM((G * 8 * BPW,), jnp.float32) for _ in range(NBUF)],
          [pltpu.SemaphoreType.DMA for _ in range(NBUF)],
          [pltpu.SemaphoreType.DMA for _ in range(NBUF)],
      ],
  )
  def gather_kernel(table_hbm, idst_hbm, out_hbm, idx_v, gbufs, tbufs,
                    gsems, ssems):
    wid = lax.axis_index("s") * NC + lax.axis_index("c")
    pltpu.sync_copy(idst_hbm.at[:, pl.ds(wid * BPW, BPW)], idx_v)

    lane = lax.iota(jnp.int32, 16)

    def fire(s, b):
      pltpu.async_copy(table_hbm.at[idx_v.at[s]], gbufs[b], gsems[b])

    def drain_gather(b):
      pltpu.make_async_copy(
          table_hbm.at[pl.ds(0, BPW)], gbufs[b], gsems[b]
      ).wait()

    def transpose(b):
      gb, tb = gbufs[b], tbufs[b]
      for d in range(LATENT):
        q_vec = lax.rem(lane + d, jnp.int32(LATENT))
        flat_base = q_vec * BPW + lane
        vals = [
            plsc.load_gather(gb, [lane + 16 * k, q_vec])
            for k in range(BPW // 16)
        ]
        for k in range(BPW // 16):
          plsc.store_scatter(tb, [flat_base + 16 * k], vals[k])

    def fire_store(s, b):
      for g in range(G):
        pltpu.async_copy(tbufs[b].at[pl.ds(8 * BPW * g, 8 * BPW)],
                         out_hbm.at[s, g, wid], ssems[b])

    def drain_store(b):
      for g in range(G):
        pltpu.make_async_copy(
            out_hbm.at[0, g, wid],
            tbufs[b].at[pl.ds(8 * BPW * g, 8 * BPW)], ssems[b]
        ).wait()

    for b in range(NBUF):
      fire(b, b)

    for b in range(NBUF):
      drain_gather(b)
      transpose(b)
      fire_store(b, b)
      fire(b + NBUF, b)

    def block(blk, _):
      for b in range(NBUF):
        s = blk * NBUF + b
        drain_gather(b)
        drain_store(b)
        transpose(b)
        fire_store(s, b)
        fire(s + NBUF, b)
      return _

    lax.fori_loop(1, NBLK - 1, block, None)

    for b in range(NBUF):
      s = (NBLK - 1) * NBUF + b
      drain_gather(b)
      drain_store(b)
      transpose(b)
      fire_store(s, b)

    for b in range(NBUF):
      drain_store(b)

  return gather_kernel


_gather = _make_gather()

TCOLS = 7813
VFULL = 7812 * 128
ROWS_OUT = TCOLS * 32
BLK_PW = 7812 // NW
XTRA = 7812 - BLK_PW * NW


def _make_detile():
  mesh = plsc.VectorSubcoreMesh(core_axis_name="c", subcore_axis_name="s")

  @functools.partial(
      pl.kernel,
      out_type=jax.ShapeDtypeStruct((ROWS_OUT, 128), jnp.float32),
      mesh=mesh,
      compiler_params=pltpu.CompilerParams(
          use_tc_tiling_on_sc=True,
          needs_layout_passes=False,
          disable_bounds_checks=True,
      ),
      scratch_types=[
          [pltpu.VMEM((LATENT, 128), jnp.float32) for _ in range(NBUF)],
          [pltpu.VMEM((LATENT, 128), jnp.float32) for _ in range(NBUF)],
          pltpu.VMEM((16, 128), jnp.float32),
          [pltpu.SemaphoreType.DMA for _ in range(NBUF)],
          [pltpu.SemaphoreType.DMA for _ in range(NBUF)],
      ],
  )
  def detile_kernel(tt_hbm, tail_hbm, out_hbm, ibufs, obufs, tailbuf,
                    isems, osems):
    wid = lax.axis_index("s") * NC + lax.axis_index("c")
    lane = lax.iota(jnp.int32, 16)

    def fire_in(c, b):
      pltpu.async_copy(tt_hbm.at[:, pl.ds(c * 128, 128)], ibufs[b], isems[b])

    def drain_in(b):
      pltpu.make_async_copy(
          tt_hbm.at[:, pl.ds(0, 128)], ibufs[b], isems[b]
      ).wait()

    def fire_out(c, b):
      pltpu.async_copy(obufs[b], out_hbm.at[pl.ds(c * 32, 32)], osems[b])

    def drain_out(b):
      pltpu.make_async_copy(
          out_hbm.at[pl.ds(0, 32)], obufs[b], osems[b]
      ).wait()

    def transpose(b):
      ib, ob = ibufs[b], obufs[b]
      for d in range(LATENT):
        q_vec = lax.rem(lane + d, jnp.int32(LATENT))
        vals = []
        for k in range(8):
          e_vec = lane + 16 * k
          vals.append(plsc.load_gather(ib, [q_vec, e_vec]))
        for k in range(8):
          e_vec = lane + 16 * k
          row_vec = lax.shift_right_logical(e_vec, 2)
          col_vec = lax.bitwise_or(
              lax.shift_left(lax.bitwise_and(e_vec, 3), 5), q_vec)
          plsc.store_scatter(ob, [row_vec, col_vec], vals[k])

    for b in range(NBUF):
      fire_in(wid + NW * b, b)

    def block(t, _):
      for b in range(NBUF):
        idx = t * NBUF + b
        drain_in(b)

        @pl.when(idx >= NBUF)
        def _():
          drain_out(b)
        transpose(b)
        fire_out(wid + NW * idx, b)

        @pl.when(idx + NBUF < BLK_PW)
        def _():
          fire_in(wid + NW * (idx + NBUF), b)
      return _

    lax.fori_loop(0, BLK_PW // NBUF, block, None, unroll=False)
    for b in range(NBUF):
      drain_out(b)

    @pl.when(wid < XTRA)
    def _():
      c = 7808 + wid
      fire_in(c, 0)
      drain_in(0)
      transpose(0)
      fire_out(c, 0)
      drain_out(0)

    @pl.when(wid == NW - 1)
    def _():
      pltpu.sync_copy(tail_hbm, tailbuf)
      pltpu.sync_copy(tailbuf, out_hbm.at[pl.ds(7812 * 32, 16)])

  return detile_kernel


_detile = _make_detile()


@jax.jit
def kernel(input_ids, table):
  ids_t = input_ids.T.astype(jnp.int32)
  tail = table[VFULL:].reshape(16, 128)
  lin = _detile(table.T, tail).reshape(TCOLS * 128, LATENT)
  out5 = _gather(lin, ids_t).reshape(SEQ, G, NW, 8, BPW)
  out = out5.transpose(2, 4, 0, 1, 3).reshape(BATCH, SEQ, LATENT)
  return out


# speedup vs baseline: 3.8667x; 2.1162x over previous
import functools

import jax
import jax.numpy as jnp
from jax import lax
from jax.experimental import pallas as pl
from jax.experimental.pallas import tpu as pltpu
from jax.experimental.pallas import tpu_sc as plsc

VOCAB = 1000000
LATENT = 32
BATCH = 4096
SEQ = 200

NC = 2
NS = 16
NW = NC * NS

BPW = BATCH // NW
G = LATENT // 8
NBUF = 2
NBLK = SEQ // NBUF


def _make_gather():
  mesh = plsc.VectorSubcoreMesh(core_axis_name="c", subcore_axis_name="s")

  @functools.partial(
      pl.kernel,
      out_type=jax.ShapeDtypeStruct((SEQ, G, NW, 8 * BPW), jnp.float32),
      mesh=mesh,
      compiler_params=pltpu.CompilerParams(
          use_tc_tiling_on_sc=False,
          needs_layout_passes=False,
          disable_bounds_checks=True,
      ),
      scratch_types=[
          pltpu.VMEM((SEQ, BPW), jnp.int32),
          [pltpu.VMEM((BPW, LATENT), jnp.float32) for _ in range(NBUF)],
          [pltpu.VMEM((G * 8 * BPW,), jnp.float32) for _ in range(NBUF)],
          [pltpu.SemaphoreType.DMA for _ in range(NBUF)],
          [pltpu.SemaphoreType.DMA for _ in range(NBUF)],
      ],
  )
  def gather_kernel(table_hbm, idst_hbm, out_hbm, idx_v, gbufs, tbufs,
                    gsems, ssems):
    wid = lax.axis_index("s") * NC + lax.axis_index("c")
    pltpu.sync_copy(idst_hbm.at[:, pl.ds(wid * BPW, BPW)], idx_v)

    lane = lax.iota(jnp.int32, 16)

    def fire(s, b):
      pltpu.async_copy(table_hbm.at[idx_v.at[s]], gbufs[b], gsems[b])

    def drain_gather(b):
      pltpu.make_async_copy(
          table_hbm.at[pl.ds(0, BPW)], gbufs[b], gsems[b]
      ).wait()

    def transpose(b):
      gb, tb = gbufs[b], tbufs[b]

      @plsc.parallel_loop(0, LATENT, unroll=4)
      def _(d):
        q_vec = lax.rem(lane + d, jnp.int32(LATENT))
        flat_base = q_vec * BPW + lane
        vals = [
            plsc.load_gather(gb, [lane + 16 * k, q_vec])
            for k in range(BPW // 16)
        ]
        for k in range(BPW // 16):
          plsc.store_scatter(tb, [flat_base + 16 * k], vals[k])

    def fire_store(s, b):
      for g in range(G):
        pltpu.async_copy(tbufs[b].at[pl.ds(8 * BPW * g, 8 * BPW)],
                         out_hbm.at[s, g, wid], ssems[b])

    def drain_store(b):
      for g in range(G):
        pltpu.make_async_copy(
            out_hbm.at[0, g, wid],
            tbufs[b].at[pl.ds(8 * BPW * g, 8 * BPW)], ssems[b]
        ).wait()

    for b in range(NBUF):
      fire(b, b)

    for b in range(NBUF):
      drain_gather(b)
      transpose(b)
      fire_store(b, b)
      fire(b + NBUF, b)

    def block(blk, _):
      for b in range(NBUF):
        s = blk * NBUF + b
        drain_gather(b)
        drain_store(b)
        transpose(b)
        fire_store(s, b)
        fire(s + NBUF, b)
      return _

    lax.fori_loop(1, NBLK - 1, block, None)

    for b in range(NBUF):
      s = (NBLK - 1) * NBUF + b
      drain_gather(b)
      drain_store(b)
      transpose(b)
      fire_store(s, b)

    for b in range(NBUF):
      drain_store(b)

  return gather_kernel


_gather = _make_gather()

TCOLS = 7813
VFULL = 7812 * 128
ROWS_OUT = TCOLS * 32
BLK_PW = 7812 // NW
XTRA = 7812 - BLK_PW * NW


def _make_detile():
  mesh = plsc.VectorSubcoreMesh(core_axis_name="c", subcore_axis_name="s")

  @functools.partial(
      pl.kernel,
      out_type=jax.ShapeDtypeStruct((ROWS_OUT, 128), jnp.float32),
      mesh=mesh,
      compiler_params=pltpu.CompilerParams(
          use_tc_tiling_on_sc=True,
          needs_layout_passes=False,
          disable_bounds_checks=True,
      ),
      scratch_types=[
          [pltpu.VMEM((LATENT, 128), jnp.float32) for _ in range(NBUF)],
          [pltpu.VMEM((LATENT, 128), jnp.float32) for _ in range(NBUF)],
          pltpu.VMEM((16, 128), jnp.float32),
          [pltpu.SemaphoreType.DMA for _ in range(NBUF)],
          [pltpu.SemaphoreType.DMA for _ in range(NBUF)],
      ],
  )
  def detile_kernel(tt_hbm, tail_hbm, out_hbm, ibufs, obufs, tailbuf,
                    isems, osems):
    wid = lax.axis_index("s") * NC + lax.axis_index("c")
    lane = lax.iota(jnp.int32, 16)

    def fire_in(c, b):
      pltpu.async_copy(tt_hbm.at[:, pl.ds(c * 128, 128)], ibufs[b], isems[b])

    def drain_in(b):
      pltpu.make_async_copy(
          tt_hbm.at[:, pl.ds(0, 128)], ibufs[b], isems[b]
      ).wait()

    def fire_out(c, b):
      pltpu.async_copy(obufs[b], out_hbm.at[pl.ds(c * 32, 32)], osems[b])

    def drain_out(b):
      pltpu.make_async_copy(
          out_hbm.at[pl.ds(0, 32)], obufs[b], osems[b]
      ).wait()

    def transpose(b):
      ib, ob = ibufs[b], obufs[b]

      @plsc.parallel_loop(0, LATENT, unroll=4)
      def _(d):
        q_vec = lax.rem(lane + d, jnp.int32(LATENT))
        vals = []
        for k in range(8):
          e_vec = lane + 16 * k
          vals.append(plsc.load_gather(ib, [q_vec, e_vec]))
        for k in range(8):
          e_vec = lane + 16 * k
          row_vec = lax.shift_right_logical(e_vec, 2)
          col_vec = lax.bitwise_or(
              lax.shift_left(lax.bitwise_and(e_vec, 3), 5), q_vec)
          plsc.store_scatter(ob, [row_vec, col_vec], vals[k])

    for b in range(NBUF):
      fire_in(wid + NW * b, b)

    def block(t, _):
      for b in range(NBUF):
        idx = t * NBUF + b
        drain_in(b)

        @pl.when(idx >= NBUF)
        def _():
          drain_out(b)
        transpose(b)
        fire_out(wid + NW * idx, b)

        @pl.when(idx + NBUF < BLK_PW)
        def _():
          fire_in(wid + NW * (idx + NBUF), b)
      return _

    lax.fori_loop(0, BLK_PW // NBUF, block, None, unroll=False)
    for b in range(NBUF):
      drain_out(b)

    @pl.when(wid < XTRA)
    def _():
      c = 7808 + wid
      fire_in(c, 0)
      drain_in(0)
      transpose(0)
      fire_out(c, 0)
      drain_out(0)

    @pl.when(wid == NW - 1)
    def _():
      pltpu.sync_copy(tail_hbm, tailbuf)
      pltpu.sync_copy(tailbuf, out_hbm.at[pl.ds(7812 * 32, 16)])

  return detile_kernel


_detile = _make_detile()


@jax.jit
def kernel(input_ids, table):
  ids_t = input_ids.T.astype(jnp.int32)
  tail = table[VFULL:].reshape(16, 128)
  lin = _detile(table.T, tail).reshape(TCOLS * 128, LATENT)
  out5 = _gather(lin, ids_t).reshape(SEQ, G, NW, 8, BPW)
  out = out5.transpose(2, 4, 0, 1, 3).reshape(BATCH, SEQ, LATENT)
  return out


# speedup vs baseline: 5.6817x; 1.4694x over previous
import functools

import jax
import jax.numpy as jnp
from jax import lax
from jax.experimental import pallas as pl
from jax.experimental.pallas import tpu as pltpu
from jax.experimental.pallas import tpu_sc as plsc

VOCAB = 1000000
LATENT = 32
BATCH = 4096
SEQ = 200

NC = 2
NS = 16
NW = NC * NS

BPW = BATCH // NW
G = LATENT // 8
NBUF = 4
NBLK = SEQ // NBUF


def _make_gather():
  mesh = plsc.VectorSubcoreMesh(core_axis_name="c", subcore_axis_name="s")

  @functools.partial(
      pl.kernel,
      out_type=jax.ShapeDtypeStruct((SEQ, G, NW, 8 * BPW), jnp.float32),
      mesh=mesh,
      compiler_params=pltpu.CompilerParams(
          use_tc_tiling_on_sc=False,
          needs_layout_passes=False,
          disable_bounds_checks=True,
      ),
      scratch_types=[
          pltpu.VMEM((SEQ, BPW), jnp.int32),
          [pltpu.VMEM((BPW, LATENT), jnp.float32) for _ in range(NBUF)],
          [pltpu.VMEM((G * 8 * BPW,), jnp.float32) for _ in range(NBUF)],
          [pltpu.SemaphoreType.DMA for _ in range(NBUF)],
          [pltpu.SemaphoreType.DMA for _ in range(NBUF)],
      ],
  )
  def gather_kernel(table_hbm, idst_hbm, out_hbm, idx_v, gbufs, tbufs,
                    gsems, ssems):
    wid = lax.axis_index("s") * NC + lax.axis_index("c")
    pltpu.sync_copy(idst_hbm.at[:, pl.ds(wid * BPW, BPW)], idx_v)

    lane = lax.iota(jnp.int32, 16)

    def fire(s, b):
      pltpu.async_copy(table_hbm.at[idx_v.at[s]], gbufs[b], gsems[b])

    def drain_gather(b):
      pltpu.make_async_copy(
          table_hbm.at[pl.ds(0, BPW)], gbufs[b], gsems[b]
      ).wait()

    def transpose(b):
      gb, tb = gbufs[b], tbufs[b]

      @plsc.parallel_loop(0, LATENT, unroll=8)
      def _(d):
        q_vec = lax.rem(lane + d, jnp.int32(LATENT))
        flat_base = q_vec * BPW + lane
        vals = [
            plsc.load_gather(gb, [lane + 16 * k, q_vec])
            for k in range(BPW // 16)
        ]
        for k in range(BPW // 16):
          plsc.store_scatter(tb, [flat_base + 16 * k], vals[k])

    def fire_store(s, b):
      for g in range(G):
        pltpu.async_copy(tbufs[b].at[pl.ds(8 * BPW * g, 8 * BPW)],
                         out_hbm.at[s, g, wid], ssems[b])

    def drain_store(b):
      for g in range(G):
        pltpu.make_async_copy(
            out_hbm.at[0, g, wid],
            tbufs[b].at[pl.ds(8 * BPW * g, 8 * BPW)], ssems[b]
        ).wait()

    for b in range(NBUF):
      fire(b, b)

    for b in range(NBUF):
      drain_gather(b)
      transpose(b)
      fire_store(b, b)
      fire(b + NBUF, b)

    def block(blk, _):
      for b in range(NBUF):
        s = blk * NBUF + b
        drain_gather(b)
        drain_store(b)
        transpose(b)
        fire_store(s, b)
        fire(s + NBUF, b)
      return _

    lax.fori_loop(1, NBLK - 1, block, None)

    for b in range(NBUF):
      s = (NBLK - 1) * NBUF + b
      drain_gather(b)
      drain_store(b)
      transpose(b)
      fire_store(s, b)

    for b in range(NBUF):
      drain_store(b)

  return gather_kernel


_gather = _make_gather()

TCOLS = 7813
VFULL = 7812 * 128
ROWS_OUT = TCOLS * 32
BLK_PW = 7812 // NW
XTRA = 7812 - BLK_PW * NW


def _make_detile():
  mesh = plsc.VectorSubcoreMesh(core_axis_name="c", subcore_axis_name="s")

  @functools.partial(
      pl.kernel,
      out_type=jax.ShapeDtypeStruct((ROWS_OUT, 128), jnp.float32),
      mesh=mesh,
      compiler_params=pltpu.CompilerParams(
          use_tc_tiling_on_sc=True,
          needs_layout_passes=False,
          disable_bounds_checks=True,
      ),
      scratch_types=[
          [pltpu.VMEM((LATENT, 128), jnp.float32) for _ in range(NBUF)],
          [pltpu.VMEM((LATENT, 128), jnp.float32) for _ in range(NBUF)],
          pltpu.VMEM((16, 128), jnp.float32),
          [pltpu.SemaphoreType.DMA for _ in range(NBUF)],
          [pltpu.SemaphoreType.DMA for _ in range(NBUF)],
      ],
  )
  def detile_kernel(tt_hbm, tail_hbm, out_hbm, ibufs, obufs, tailbuf,
                    isems, osems):
    wid = lax.axis_index("s") * NC + lax.axis_index("c")
    lane = lax.iota(jnp.int32, 16)

    def fire_in(c, b):
      pltpu.async_copy(tt_hbm.at[:, pl.ds(c * 128, 128)], ibufs[b], isems[b])

    def drain_in(b):
      pltpu.make_async_copy(
          tt_hbm.at[:, pl.ds(0, 128)], ibufs[b], isems[b]
      ).wait()

    def fire_out(c, b):
      pltpu.async_copy(obufs[b], out_hbm.at[pl.ds(c * 32, 32)], osems[b])

    def drain_out(b):
      pltpu.make_async_copy(
          out_hbm.at[pl.ds(0, 32)], obufs[b], osems[b]
      ).wait()

    def transpose(b):
      ib, ob = ibufs[b], obufs[b]

      @plsc.parallel_loop(0, LATENT, unroll=8)
      def _(d):
        q_vec = lax.rem(lane + d, jnp.int32(LATENT))
        vals = []
        for k in range(8):
          e_vec = lane + 16 * k
          vals.append(plsc.load_gather(ib, [q_vec, e_vec]))
        for k in range(8):
          e_vec = lane + 16 * k
          row_vec = lax.shift_right_logical(e_vec, 2)
          col_vec = lax.bitwise_or(
              lax.shift_left(lax.bitwise_and(e_vec, 3), 5), q_vec)
          plsc.store_scatter(ob, [row_vec, col_vec], vals[k])

    for b in range(NBUF):
      fire_in(wid + NW * b, b)

    def block(t, _):
      for b in range(NBUF):
        idx = t * NBUF + b
        drain_in(b)

        @pl.when(idx >= NBUF)
        def _():
          drain_out(b)
        transpose(b)
        fire_out(wid + NW * idx, b)

        @pl.when(idx + NBUF < BLK_PW)
        def _():
          fire_in(wid + NW * (idx + NBUF), b)
      return _

    lax.fori_loop(0, BLK_PW // NBUF, block, None, unroll=False)
    for b in range(NBUF):
      drain_out(b)

    @pl.when(wid < XTRA)
    def _():
      c = 7808 + wid
      fire_in(c, 0)
      drain_in(0)
      transpose(0)
      fire_out(c, 0)
      drain_out(0)

    @pl.when(wid == NW - 1)
    def _():
      pltpu.sync_copy(tail_hbm, tailbuf)
      pltpu.sync_copy(tailbuf, out_hbm.at[pl.ds(7812 * 32, 16)])

  return detile_kernel


_detile = _make_detile()


@jax.jit
def kernel(input_ids, table):
  ids_t = input_ids.T.astype(jnp.int32)
  tail = table[VFULL:].reshape(16, 128)
  lin = _detile(table.T, tail).reshape(TCOLS * 128, LATENT)
  out5 = _gather(lin, ids_t).reshape(SEQ, G, NW, 8, BPW)
  out = out5.transpose(2, 4, 0, 1, 3).reshape(BATCH, SEQ, LATENT)
  return out


# speedup vs baseline: 5.6878x; 1.0011x over previous
import functools

import jax
import jax.numpy as jnp
from jax import lax
from jax.experimental import pallas as pl
from jax.experimental.pallas import tpu as pltpu
from jax.experimental.pallas import tpu_sc as plsc

VOCAB = 1000000
LATENT = 32
BATCH = 4096
SEQ = 200

NC = 2
NS = 16
NW = NC * NS

BPW = BATCH // NW
G = LATENT // 8
NBUF = 4
NBUF_G = 8
NBLK = SEQ // NBUF_G


def _make_gather():
  mesh = plsc.VectorSubcoreMesh(core_axis_name="c", subcore_axis_name="s")

  @functools.partial(
      pl.kernel,
      out_type=jax.ShapeDtypeStruct((SEQ, G, NW, 8 * BPW), jnp.float32),
      mesh=mesh,
      compiler_params=pltpu.CompilerParams(
          use_tc_tiling_on_sc=False,
          needs_layout_passes=False,
          disable_bounds_checks=True,
      ),
      scratch_types=[
          pltpu.VMEM((SEQ, BPW), jnp.int32),
          [pltpu.VMEM((BPW, LATENT), jnp.float32) for _ in range(NBUF_G)],
          [pltpu.VMEM((G * 8 * BPW,), jnp.float32) for _ in range(NBUF_G)],
          [pltpu.SemaphoreType.DMA for _ in range(NBUF_G)],
          [pltpu.SemaphoreType.DMA for _ in range(NBUF_G)],
      ],
  )
  def gather_kernel(table_hbm, idst_hbm, out_hbm, idx_v, gbufs, tbufs,
                    gsems, ssems):
    wid = lax.axis_index("s") * NC + lax.axis_index("c")
    pltpu.sync_copy(idst_hbm.at[:, pl.ds(wid * BPW, BPW)], idx_v)

    lane = lax.iota(jnp.int32, 16)

    def fire(s, b):
      pltpu.async_copy(table_hbm.at[idx_v.at[s]], gbufs[b], gsems[b])

    def drain_gather(b):
      pltpu.make_async_copy(
          table_hbm.at[pl.ds(0, BPW)], gbufs[b], gsems[b]
      ).wait()

    def transpose(b):
      gb, tb = gbufs[b], tbufs[b]

      @plsc.parallel_loop(0, LATENT, unroll=8)
      def _(d):
        q_vec = lax.rem(lane + d, jnp.int32(LATENT))
        flat_base = q_vec * BPW + lane
        vals = [
            plsc.load_gather(gb, [lane + 16 * k, q_vec])
            for k in range(BPW // 16)
        ]
        for k in range(BPW // 16):
          plsc.store_scatter(tb, [flat_base + 16 * k], vals[k])

    def fire_store(s, b):
      for g in range(G):
        pltpu.async_copy(tbufs[b].at[pl.ds(8 * BPW * g, 8 * BPW)],
                         out_hbm.at[s, g, wid], ssems[b])

    def drain_store(b):
      for g in range(G):
        pltpu.make_async_copy(
            out_hbm.at[0, g, wid],
            tbufs[b].at[pl.ds(8 * BPW * g, 8 * BPW)], ssems[b]
        ).wait()

    for b in range(NBUF_G):
      fire(b, b)

    for b in range(NBUF_G):
      drain_gather(b)
      transpose(b)
      fire_store(b, b)
      fire(b + NBUF_G, b)

    def block(blk, _):
      for b in range(NBUF_G):
        s = blk * NBUF_G + b
        drain_gather(b)
        drain_store(b)
        transpose(b)
        fire_store(s, b)
        fire(s + NBUF_G, b)
      return _

    lax.fori_loop(1, NBLK - 1, block, None)

    for b in range(NBUF_G):
      s = (NBLK - 1) * NBUF_G + b
      drain_gather(b)
      drain_store(b)
      transpose(b)
      fire_store(s, b)

    for b in range(NBUF_G):
      drain_store(b)

  return gather_kernel


_gather = _make_gather()

TCOLS = 7813
VFULL = 7812 * 128
ROWS_OUT = TCOLS * 32
BLK_PW = 7812 // NW
XTRA = 7812 - BLK_PW * NW


def _make_detile():
  mesh = plsc.VectorSubcoreMesh(core_axis_name="c", subcore_axis_name="s")

  @functools.partial(
      pl.kernel,
      out_type=jax.ShapeDtypeStruct((ROWS_OUT, 128), jnp.float32),
      mesh=mesh,
      compiler_params=pltpu.CompilerParams(
          use_tc_tiling_on_sc=True,
          needs_layout_passes=False,
          disable_bounds_checks=True,
      ),
      scratch_types=[
          [pltpu.VMEM((LATENT, 128), jnp.float32) for _ in range(NBUF)],
          [pltpu.VMEM((LATENT, 128), jnp.float32) for _ in range(NBUF)],
          pltpu.VMEM((16, 128), jnp.float32),
          [pltpu.SemaphoreType.DMA for _ in range(NBUF)],
          [pltpu.SemaphoreType.DMA for _ in range(NBUF)],
      ],
  )
  def detile_kernel(tt_hbm, tail_hbm, out_hbm, ibufs, obufs, tailbuf,
                    isems, osems):
    wid = lax.axis_index("s") * NC + lax.axis_index("c")
    lane = lax.iota(jnp.int32, 16)

    def fire_in(c, b):
      pltpu.async_copy(tt_hbm.at[:, pl.ds(c * 128, 128)], ibufs[b], isems[b])

    def drain_in(b):
      pltpu.make_async_copy(
          tt_hbm.at[:, pl.ds(0, 128)], ibufs[b], isems[b]
      ).wait()

    def fire_out(c, b):
      pltpu.async_copy(obufs[b], out_hbm.at[pl.ds(c * 32, 32)], osems[b])

    def drain_out(b):
      pltpu.make_async_copy(
          out_hbm.at[pl.ds(0, 32)], obufs[b], osems[b]
      ).wait()

    def transpose(b):
      ib, ob = ibufs[b], obufs[b]

      @plsc.parallel_loop(0, LATENT, unroll=8)
      def _(d):
        q_vec = lax.rem(lane + d, jnp.int32(LATENT))
        vals = []
        for k in range(8):
          e_vec = lane + 16 * k
          vals.append(plsc.load_gather(ib, [q_vec, e_vec]))
        for k in range(8):
          e_vec = lane + 16 * k
          row_vec = lax.shift_right_logical(e_vec, 2)
          col_vec = lax.bitwise_or(
              lax.shift_left(lax.bitwise_and(e_vec, 3), 5), q_vec)
          plsc.store_scatter(ob, [row_vec, col_vec], vals[k])

    for b in range(NBUF):
      fire_in(wid + NW * b, b)

    def block(t, _):
      for b in range(NBUF):
        idx = t * NBUF + b
        drain_in(b)

        @pl.when(idx >= NBUF)
        def _():
          drain_out(b)
        transpose(b)
        fire_out(wid + NW * idx, b)

        @pl.when(idx + NBUF < BLK_PW)
        def _():
          fire_in(wid + NW * (idx + NBUF), b)
      return _

    lax.fori_loop(0, BLK_PW // NBUF, block, None, unroll=False)
    for b in range(NBUF):
      drain_out(b)

    @pl.when(wid < XTRA)
    def _():
      c = 7808 + wid
      fire_in(c, 0)
      drain_in(0)
      transpose(0)
      fire_out(c, 0)
      drain_out(0)

    @pl.when(wid == NW - 1)
    def _():
      pltpu.sync_copy(tail_hbm, tailbuf)
      pltpu.sync_copy(tailbuf, out_hbm.at[pl.ds(7812 * 32, 16)])

  return detile_kernel


_detile = _make_detile()


@jax.jit
def kernel(input_ids, table):
  ids_t = input_ids.T.astype(jnp.int32)
  tail = table[VFULL:].reshape(16, 128)
  lin = _detile(table.T, tail).reshape(TCOLS * 128, LATENT)
  out5 = _gather(lin, ids_t).reshape(SEQ, G, NW, 8, BPW)
  out = out5.transpose(2, 4, 0, 1, 3).reshape(BATCH, SEQ, LATENT)
  return out
